# Initial kernel scaffold; baseline (speedup 1.0000x reference)
import jax
import jax.numpy as jnp
from jax.experimental import pallas as pl


def kernel(out_img, tgt_img, w1, b1, w2, b2, bins):
    raise NotImplementedError("write your pallas kernel here")


# speedup vs baseline: 1.1180x; 1.1180x over previous
import functools

import jax
import jax.numpy as jnp
from jax.experimental import pallas as pl
from jax.experimental.pallas import tpu as pltpu


def _pool_body(out_ref, tgt_ref, sum_f_ref, sum_t_ref):
    sum_f_ref[...] = jnp.sum(out_ref[...], axis=1, keepdims=True)
    sum_t_ref[...] = jnp.sum(tgt_ref[...], axis=1, keepdims=True)


def _head_body(sum_f_ref, sum_t_ref, w1_ref, b1_ref, w2_ref, b2_ref,
               bins_ref, res_ref, *, n, c, inv_hw):
    inv_n = 1.0 / float(n)

    def mean_score(row_sums):
        pooled = row_sums.reshape(n, c) * inv_hw
        h = jnp.dot(pooled, w1_ref[...],
                    preferred_element_type=jnp.float32) + b1_ref[...]
        h = jnp.maximum(h, 0.0)
        logits = jnp.dot(h, w2_ref[...],
                         preferred_element_type=jnp.float32) + b2_ref[...]
        m = jnp.max(logits, axis=-1, keepdims=True)
        e = jnp.exp(logits - m)
        p = e / jnp.sum(e, axis=-1, keepdims=True)
        scores = jnp.sum(p * bins_ref[...], axis=-1)
        return jnp.sum(scores) * inv_n

    res_ref[0, 0] = jnp.abs(mean_score(sum_t_ref[...]) -
                            mean_score(sum_f_ref[...]))


def kernel(out_img, tgt_img, w1, b1, w2, b2, bins):
    N, C, H, W = out_img.shape
    HW = H * W
    NC = N * C
    itemsize = jnp.dtype(out_img.dtype).itemsize

    target = (4 << 20) // max(1, HW * itemsize)
    if NC % 16 == 0:
        blk = 8
        while 2 * blk <= target and NC % (4 * blk) == 0:
            blk *= 2
        grid = (2, NC // (2 * blk))
    else:
        blk = NC
        grid = (1, 1)
    spc = grid[1]

    out2d = out_img.reshape(NC, HW)
    tgt2d = tgt_img.reshape(NC, HW)

    img_spec = pl.BlockSpec((blk, HW), lambda cidx, i: (cidx * spc + i, 0))
    sum_spec = pl.BlockSpec((blk, 1), lambda cidx, i: (cidx * spc + i, 0))

    bytes_streamed = 2 * NC * HW * itemsize
    sum_f, sum_t = pl.pallas_call(
        _pool_body,
        out_shape=(jax.ShapeDtypeStruct((NC, 1), jnp.float32),
                   jax.ShapeDtypeStruct((NC, 1), jnp.float32)),
        grid=grid,
        in_specs=[img_spec, img_spec],
        out_specs=(sum_spec, sum_spec),
        compiler_params=pltpu.CompilerParams(
            dimension_semantics=("parallel", "arbitrary"),
            vmem_limit_bytes=64 * 1024 * 1024),
        cost_estimate=pl.CostEstimate(
            flops=2 * NC * HW,
            transcendentals=0,
            bytes_accessed=bytes_streamed + 2 * NC * 4),
    )(out2d, tgt2d)

    res = pl.pallas_call(
        functools.partial(_head_body, n=N, c=C, inv_hw=1.0 / float(HW)),
        out_shape=jax.ShapeDtypeStruct((1, 1), jnp.float32),
        in_specs=[pl.BlockSpec(memory_space=pltpu.MemorySpace.VMEM)] * 7,
        out_specs=pl.BlockSpec(memory_space=pltpu.MemorySpace.SMEM),
    )(sum_f, sum_t, w1, b1, w2, b2, bins)
    return res[0, 0]
